# baseline (speedup 1.0000x reference)
import functools

import jax
import jax.numpy as jnp
from jax import lax
from jax.experimental import pallas as pl
from jax.experimental.pallas import tpu as pltpu
from jax.experimental.pallas import tpu_sc as plsc

D = 32
NW = 32
C = 128
S = 200
SB = 25
JB = 32


def _transpose_block(src, dst):
    row_vecs = [lax.iota(jnp.int32, 16) + (g * 16) for g in range(8)]
    for d in range(D):
        col = jnp.full((16,), d, dtype=jnp.int32)
        row = dst.at[d]
        for g in range(8):
            v = plsc.load_gather(src, [row_vecs[g], col])
            row[pl.ds(g * 16, 16)] = v


@functools.lru_cache(maxsize=None)
def _make_kernel():
    mesh = plsc.VectorSubcoreMesh(core_axis_name="c", subcore_axis_name="s")

    @functools.partial(
        pl.kernel,
        mesh=mesh,
        compiler_params=pltpu.CompilerParams(
            use_tc_tiling_on_sc=False, needs_layout_passes=False),
        out_type=(
            jax.ShapeDtypeStruct((S, 4, JB, 8, C), jnp.float32),
            jax.ShapeDtypeStruct((S, 4, JB, 8, C), jnp.float32),
        ),
        scratch_types=[
            pltpu.VMEM((SB, 8, C), jnp.int32),
            pltpu.VMEM((2, C, D), jnp.float32),
            pltpu.VMEM((2, C, D), jnp.float32),
            pltpu.VMEM((2, D, C), jnp.float32),
            pltpu.VMEM((2, D, C), jnp.float32),
            pltpu.SemaphoreType.DMA,
            pltpu.SemaphoreType.DMA,
            pltpu.SemaphoreType.DMA,
            pltpu.SemaphoreType.DMA,
        ],
    )
    def k(ids_hbm, wre_hbm, wim_hbm, ore_hbm, oim_hbm,
          idx_v, bre, bim, tre, tim, sem_g0, sem_g1, sem_w0, sem_w1):
        j = lax.axis_index("s") * 2 + lax.axis_index("c")
        for ti in range(SB):
            pltpu.sync_copy(ids_hbm.at[ti, j], idx_v.at[ti])

        def fire_gathers(s, buf_set, sem):
            idx = idx_v.at[s // 8, s % 8]
            return (
                pltpu.async_copy(wre_hbm.at[idx], bre.at[buf_set], sem),
                pltpu.async_copy(wim_hbm.at[idx], bim.at[buf_set], sem),
            )

        def drain_writes(buf_set, sem):
            for _ in range(8):
                pltpu.make_async_copy(
                    tre.at[buf_set, pl.ds(0, 8)], ore_hbm.at[0, 0, 0], sem
                ).wait()

        def finish_chunk(s, buf_set, gathers, sem_w):
            for cp in gathers:
                cp.wait()
            _transpose_block(bre.at[buf_set], tre.at[buf_set])
            _transpose_block(bim.at[buf_set], tim.at[buf_set])
            for i in range(4):
                pltpu.async_copy(
                    tre.at[buf_set, pl.ds(8 * i, 8)], ore_hbm.at[s, i, j], sem_w)
                pltpu.async_copy(
                    tim.at[buf_set, pl.ds(8 * i, 8)], oim_hbm.at[s, i, j], sem_w)

        def body(t, carry):
            s0 = 2 * t
            s1 = 2 * t + 1

            @pl.when(t > 0)
            def _():
                drain_writes(0, sem_w0)
            g0 = fire_gathers(s0, 0, sem_g0)

            @pl.when(t > 0)
            def _():
                drain_writes(1, sem_w1)
            g1 = fire_gathers(s1, 1, sem_g1)

            finish_chunk(s0, 0, g0, sem_w0)
            finish_chunk(s1, 1, g1, sem_w1)
            return carry

        lax.fori_loop(0, S // 2, body, 0)
        drain_writes(0, sem_w0)
        drain_writes(1, sem_w1)

    return k


def kernel(input_ids, W_re, W_im):
    ids4 = input_ids.reshape(JB, C, SB, 8).transpose(2, 0, 3, 1)
    o5re, o5im = _make_kernel()(ids4, W_re, W_im)
    out_re = jnp.transpose(o5re, (2, 4, 0, 1, 3)).reshape(4096, S, D)
    out_im = jnp.transpose(o5im, (2, 4, 0, 1, 3)).reshape(4096, S, D)
    return (out_re, out_im)


# speedup vs baseline: 1.3993x; 1.3993x over previous
import functools

import jax
import jax.numpy as jnp
from jax import lax
from jax.experimental import pallas as pl
from jax.experimental.pallas import tpu as pltpu
from jax.experimental.pallas import tpu_sc as plsc

D = 32
NW = 32
C = 128
K = 4
CH = K * C


@functools.lru_cache(maxsize=None)
def _make_kernel(total: int):
    per_w = total // NW
    nch = per_w // C
    nit = nch // (2 * K)
    mesh = plsc.VectorSubcoreMesh(core_axis_name="c", subcore_axis_name="s")

    @functools.partial(
        pl.kernel,
        mesh=mesh,
        compiler_params=pltpu.CompilerParams(
            use_tc_tiling_on_sc=False, skip_device_barrier=True),
        out_type=(
            jax.ShapeDtypeStruct((total, D), jnp.float32),
            jax.ShapeDtypeStruct((total, D), jnp.float32),
        ),
        scratch_types=[
            pltpu.VMEM((nch, C), jnp.int32),
            pltpu.VMEM((CH, D), jnp.float32),
            pltpu.VMEM((CH, D), jnp.float32),
            pltpu.VMEM((CH, D), jnp.float32),
            pltpu.VMEM((CH, D), jnp.float32),
            pltpu.SemaphoreType.DMA,
            pltpu.SemaphoreType.DMA,
            pltpu.SemaphoreType.DMA,
            pltpu.SemaphoreType.DMA,
        ],
    )
    def k(ids_hbm, wre_hbm, wim_hbm, ore_hbm, oim_hbm,
          idx_v, bre0, bim0, bre1, bim1, sem_g0, sem_g1, sem_w0, sem_w1):
        wid = lax.axis_index("s") * 2 + lax.axis_index("c")
        pltpu.sync_copy(ids_hbm.at[wid], idx_v)
        base = wid * per_w

        def drain_writes(bre, bim, sem):
            pltpu.make_async_copy(bre, ore_hbm.at[pl.ds(0, CH)], sem).wait()
            pltpu.make_async_copy(bim, oim_hbm.at[pl.ds(0, CH)], sem).wait()

        def fire_gathers(c0, bre, bim, sem):
            cps = []
            for i in range(K):
                idx = idx_v.at[c0 + i]
                cps.append(pltpu.async_copy(
                    wre_hbm.at[idx], bre.at[pl.ds(i * C, C)], sem))
                cps.append(pltpu.async_copy(
                    wim_hbm.at[idx], bim.at[pl.ds(i * C, C)], sem))
            return cps

        def body(jj, carry):
            c0 = 2 * K * jj
            c1 = c0 + K

            @pl.when(jj > 0)
            def _():
                drain_writes(bre0, bim0, sem_w0)
            g0 = fire_gathers(c0, bre0, bim0, sem_g0)

            @pl.when(jj > 0)
            def _():
                drain_writes(bre1, bim1, sem_w1)
            g1 = fire_gathers(c1, bre1, bim1, sem_g1)

            for cp in g0:
                cp.wait()
            pltpu.async_copy(bre0, ore_hbm.at[pl.ds(base + c0 * C, CH)], sem_w0)
            pltpu.async_copy(bim0, oim_hbm.at[pl.ds(base + c0 * C, CH)], sem_w0)

            for cp in g1:
                cp.wait()
            pltpu.async_copy(bre1, ore_hbm.at[pl.ds(base + c1 * C, CH)], sem_w1)
            pltpu.async_copy(bim1, oim_hbm.at[pl.ds(base + c1 * C, CH)], sem_w1)
            return carry

        lax.fori_loop(0, nit, body, 0)
        drain_writes(bre0, bim0, sem_w0)
        drain_writes(bre1, bim1, sem_w1)

    return k


def kernel(input_ids, W_re, W_im):
    b, s = input_ids.shape
    total = b * s
    ids3 = input_ids.reshape(NW, total // NW // C, C)
    out_re, out_im = _make_kernel(total)(ids3, W_re, W_im)
    return (out_re.reshape(b, s, D), out_im.reshape(b, s, D))
